# baseline (speedup 1.0000x reference)
import functools

import jax
import jax.numpy as jnp
import numpy as np
from jax import lax
from jax.experimental import pallas as pl
from jax.experimental.pallas import tpu as pltpu
from jax.experimental.pallas import tpu_sc as plsc

_FIELD_DIMS = [1000] * 26
_NUM_FIELDS = len(_FIELD_DIMS)
_EMBED = 128
_BATCH = 16384
_OFFSETS = np.array((0, *np.cumsum(_FIELD_DIMS)[:-1]), dtype=np.int32)

_N = _BATCH * _NUM_FIELDS
_NW = 32
_PER_W = _N // _NW
_CHUNK = 128
_NCHUNK = _PER_W // _CHUNK
_LANES = 16

_OFFS_FLAT = np.repeat(_OFFSETS, _BATCH)

_mesh = plsc.VectorSubcoreMesh(core_axis_name="c", subcore_axis_name="s")


@functools.partial(
    pl.kernel,
    mesh=_mesh,
    out_type=jax.ShapeDtypeStruct((_NUM_FIELDS, _BATCH, _EMBED), jnp.float32),
    scratch_types=[
        pltpu.VMEM((_PER_W,), jnp.int32),
        pltpu.VMEM((_PER_W,), jnp.int32),
        pltpu.VMEM((6, _CHUNK, _EMBED), jnp.float32),
        pltpu.SemaphoreType.DMA,
        pltpu.SemaphoreType.DMA,
    ],
    compiler_params=pltpu.CompilerParams(use_tc_tiling_on_sc=True),
)
def _emb_lookup(x_hbm, offs_hbm, table_hbm, out_hbm, idx_v, offs_v, rows_v,
                gsem, wsem):
    wid = lax.axis_index("s") * 2 + lax.axis_index("c")
    q_base = wid * _PER_W

    pltpu.sync_copy(x_hbm.at[pl.ds(q_base, _PER_W)], idx_v)
    pltpu.sync_copy(offs_hbm.at[pl.ds(q_base, _PER_W)], offs_v)

    def _add(i, carry):
        sl = pl.ds(i * _LANES, _LANES)
        idx_v[sl] = (idx_v[sl] + offs_v[sl]) & 2047
        return carry

    lax.fori_loop(0, _PER_W // _LANES, _add, 0)

    def _dst(j):
        q0 = q_base + j * _CHUNK
        return out_hbm.at[q0 // _BATCH].at[pl.ds(q0 % _BATCH, _CHUNK)]

    def _start_gather(j, b):
        pltpu.async_copy(table_hbm.at[idx_v.at[pl.ds(j * _CHUNK, _CHUNK)]],
                         rows_v.at[b], gsem)

    def _wait_gather(j, b):
        pltpu.make_async_copy(
            table_hbm.at[idx_v.at[pl.ds(j * _CHUNK, _CHUNK)]],
            rows_v.at[b], gsem).wait()

    def _start_write(j, b):
        pltpu.async_copy(rows_v.at[b], _dst(j), wsem)

    def _wait_write(j, b):
        pltpu.make_async_copy(rows_v.at[b], _dst(j), wsem).wait()

    for j in (0, 1, 2):
        _start_gather(j, j)
    for j in (0, 1, 2):
        _wait_gather(j, j)
        _start_write(j, j)
        _start_gather(j + 3, j + 3)

    def _steady(s, carry):
        for k in range(6):
            j = 3 + s * 6 + k
            b = (3 + k) % 6
            _wait_gather(j, b)
            _start_write(j, b)
            _wait_write(j - 3, (b + 3) % 6)
            _start_gather(j + 3, (b + 3) % 6)
        return carry

    lax.fori_loop(0, (_NCHUNK - 8) // 6, _steady, 0)

    for j in range(_NCHUNK - 5, _NCHUNK):
        b = j % 6
        _wait_gather(j, b)
        _start_write(j, b)
        _wait_write(j - 3, (b + 3) % 6)
        if j + 3 < _NCHUNK:
            _start_gather(j + 3, (b + 3) % 6)
    for j in range(_NCHUNK - 3, _NCHUNK):
        _wait_write(j, j % 6)


def kernel(x, table):
    x_fmajor = x.astype(jnp.int32).T.reshape(_N)
    offs = jnp.asarray(_OFFS_FLAT, dtype=jnp.int32)
    out = _emb_lookup(x_fmajor, offs, table)
    return out.transpose(1, 0, 2)


# speedup vs baseline: 1.2473x; 1.2473x over previous
import functools

import jax
import jax.numpy as jnp
import numpy as np
from jax import lax
from jax.experimental import pallas as pl
from jax.experimental.pallas import tpu as pltpu
from jax.experimental.pallas import tpu_sc as plsc

_FIELD_DIMS = [1000] * 26
_NUM_FIELDS = len(_FIELD_DIMS)
_EMBED = 128
_BATCH = 16384
_OFFSETS = np.array((0, *np.cumsum(_FIELD_DIMS)[:-1]), dtype=np.int32)

_N = _BATCH * _NUM_FIELDS
_NW = 32
_PER_W = _N // _NW
_CHUNK = 128
_NCHUNK = _PER_W // _CHUNK
_LANES = 16

_OFFS_FLAT = np.repeat(_OFFSETS, _BATCH)

_mesh = plsc.VectorSubcoreMesh(core_axis_name="c", subcore_axis_name="s")


@functools.partial(
    pl.kernel,
    mesh=_mesh,
    out_type=jax.ShapeDtypeStruct((_NUM_FIELDS, _BATCH, _EMBED), jnp.float32),
    scratch_types=[
        pltpu.VMEM((_PER_W,), jnp.int32),
        pltpu.VMEM((_PER_W,), jnp.int32),
        pltpu.VMEM((6, _CHUNK, _EMBED), jnp.float32),
        pltpu.SemaphoreType.DMA,
        pltpu.SemaphoreType.DMA,
    ],
    compiler_params=pltpu.CompilerParams(use_tc_tiling_on_sc=True),
)
def _emb_lookup(x_hbm, offs_hbm, table_hbm, out_hbm, idx_v, offs_v, rows_v,
                gsem, wsem):
    wid = lax.axis_index("s") * 2 + lax.axis_index("c")
    q_base = wid * _PER_W

    pltpu.sync_copy(x_hbm.at[pl.ds(q_base, _PER_W)], idx_v)
    pltpu.sync_copy(offs_hbm.at[pl.ds(q_base, _PER_W)], offs_v)

    def _add_chunk(j):
        for v in range(_CHUNK // _LANES):
            sl = pl.ds(j * _CHUNK + v * _LANES, _LANES)
            idx_v[sl] = idx_v[sl] + offs_v[sl]

    def _dst(j):
        q0 = q_base + j * _CHUNK
        return out_hbm.at[q0 // _BATCH].at[pl.ds(q0 % _BATCH, _CHUNK)]

    def _start_gather(j, b):
        pltpu.async_copy(table_hbm.at[idx_v.at[pl.ds(j * _CHUNK, _CHUNK)]],
                         rows_v.at[b], gsem)

    def _wait_gather(j, b):
        pltpu.make_async_copy(
            table_hbm.at[idx_v.at[pl.ds(j * _CHUNK, _CHUNK)]],
            rows_v.at[b], gsem).wait()

    def _start_write(j, b):
        pltpu.async_copy(rows_v.at[b], _dst(j), wsem)

    def _wait_write(j, b):
        pltpu.make_async_copy(rows_v.at[b], _dst(j), wsem).wait()

    for j in range(6):
        _add_chunk(j)
    for j in (0, 1, 2):
        _start_gather(j, j)
    for j in (0, 1, 2):
        _wait_gather(j, j)
        _start_write(j, j)
        _add_chunk(j + 6)
        _start_gather(j + 3, j + 3)

    def _steady(s, carry):
        for k in range(6):
            j = 3 + s * 6 + k
            b = (3 + k) % 6
            _wait_gather(j, b)
            _start_write(j, b)
            _wait_write(j - 3, (b + 3) % 6)
            _add_chunk(j + 6)
            _start_gather(j + 3, (b + 3) % 6)
        return carry

    lax.fori_loop(0, (_NCHUNK - 14) // 6, _steady, 0)

    for j in range(_NCHUNK - 11, _NCHUNK):
        b = j % 6
        _wait_gather(j, b)
        _start_write(j, b)
        _wait_write(j - 3, (b + 3) % 6)
        if j + 6 < _NCHUNK:
            _add_chunk(j + 6)
        if j + 3 < _NCHUNK:
            _start_gather(j + 3, (b + 3) % 6)
    for j in range(_NCHUNK - 3, _NCHUNK):
        _wait_write(j, j % 6)


def kernel(x, table):
    x_fmajor = x.astype(jnp.int32).T.reshape(_N)
    offs = jnp.asarray(_OFFS_FLAT, dtype=jnp.int32)
    out = _emb_lookup(x_fmajor, offs, table)
    return out.transpose(1, 0, 2)
